# baseline (speedup 1.0000x reference)
import functools

import jax
import jax.numpy as jnp
from jax import lax
from jax.experimental import pallas as pl
from jax.experimental.pallas import tpu as pltpu
from jax.experimental.pallas import tpu_sc as plsc

H = W = 512
K = 2048
NKEY = 2 * K
NPIX = H * W
NINST = 15
NTASK = NINST

_info = plsc.get_sparse_core_info()
_NC, _NS, _L = _info.num_cores, _info.num_subcores, _info.num_lanes
_NW = _NC * _NS

SC_CHUNK = 16384
SC_NCHUNK = NPIX // SC_CHUNK


def _stage1_body(pred_ref, inst_ref, lab_ref, keys_ref, stats_ref):
    f32 = jnp.float32

    col = lax.broadcasted_iota(jnp.int32, (H, W), 1).astype(f32) * (1.0 / (W - 1))
    row = lax.broadcasted_iota(jnp.int32, (H, W), 0).astype(f32) * (1.0 / (H - 1))
    emb_x = jnp.tanh(pred_ref[0]) + col
    emb_y = jnp.tanh(pred_ref[1]) + row
    sgx = pred_ref[2]
    sgy = pred_ref[3]
    seed = 1.0 / (1.0 + jnp.exp(-pred_ref[4]))
    inst = inst_ref[...]
    lab = lab_ref[...]

    bg_seed = jnp.sum(jnp.where(lab == 0, seed * seed, 0.0))

    si = lax.broadcasted_iota(jnp.int32, (16, 128), 0)
    li = lax.broadcasted_iota(jnp.int32, (16, 128), 1)

    def body(iid, stats):
        mb = inst == iid
        mf = mb.astype(f32)
        cnt = jnp.sum(mf)
        present = cnt > 0.0
        sc = jnp.where(present, cnt, 1.0)
        cx = jnp.sum(mf * emb_x) / sc
        cy = jnp.sum(mf * emb_y) / sc
        sx = jnp.sum(mf * sgx) / sc
        sy = jnp.sum(mf * sgy) / sc
        var_term = (jnp.sum(mf * (sgx - sx) ** 2)
                    + jnp.sum(mf * (sgy - sy) ** 2)) / (2.0 * sc)
        sex = jnp.exp(10.0 * sx)
        sey = jnp.exp(10.0 * sy)
        d = jnp.exp(-((emb_x - cx) ** 2 * sex + (emb_y - cy) ** 2 * sey))
        seed_term = jnp.sum(mf * (seed - d) ** 2)
        e = 1.0 - (d * 2.0 - 1.0) * (mf * 2.0 - 1.0)
        kbin = jnp.clip((e * (K / 2.0)).astype(jnp.int32), 0, K - 1)
        key = kbin + jnp.where(mb, K, 0)
        keys_ref[pl.ds(iid - 1, 1)] = key[None]

        r = iid - 1
        stats = stats + jnp.where((si == r) & (li == 0), cnt, 0.0) \
            + jnp.where((si == r) & (li == 1), var_term, 0.0) \
            + jnp.where((si == r) & (li == 2), seed_term, 0.0)
        return stats

    stats = lax.fori_loop(1, 16, body, jnp.zeros((16, 128), f32))
    stats = stats + jnp.where((si == 0) & (li == 3), bg_seed, 0.0)
    stats_ref[...] = stats


SC_UNROLL = 16
HSTRIDE = NKEY + 1
HSIZE = HSTRIDE * _L


def _hist_body(keys_hbm, zeros_hbm, out_hbm, chunk_v, hist_v, red_v, sem0, sem1):
    wid = lax.axis_index("s") * _NC + lax.axis_index("c")
    lane = lax.iota(jnp.int32, _L)
    loff = lane * HSTRIDE
    ones = jnp.ones((_L,), jnp.float32)
    zeros16 = jnp.zeros((_L,), jnp.float32)
    sems = (sem0, sem1)

    for rnd in range((NTASK + _NW - 1) // _NW):
        task = wid + rnd * _NW

        @pl.when(task < NTASK)
        def _():
            pltpu.sync_copy(zeros_hbm, hist_v)

            cps = {}
            cps[0] = pltpu.async_copy(
                keys_hbm.at[task, pl.ds(0, SC_CHUNK)], chunk_v.at[0], sems[0])
            for ch in range(SC_NCHUNK):
                buf = ch % 2
                if ch + 1 < SC_NCHUNK:
                    nbuf = (ch + 1) % 2
                    cps[ch + 1] = pltpu.async_copy(
                        keys_hbm.at[task, pl.ds((ch + 1) * SC_CHUNK, SC_CHUNK)],
                        chunk_v.at[nbuf], sems[nbuf])
                cps[ch].wait()

                @plsc.parallel_loop(0, SC_CHUNK // _L, 1, unroll=SC_UNROLL)
                def _scatter(j):
                    kk = chunk_v[buf, pl.ds(j * _L, _L)]
                    plsc.addupdate_scatter(hist_v, [kk + loff], ones)

            @plsc.parallel_loop(0, NKEY // _L, 1, unroll=2)
            def _reduce(j):
                acc = zeros16
                for ln in range(_L):
                    iv = lane + (ln * HSTRIDE + j * _L)
                    acc = acc + plsc.load_gather(hist_v, [iv])
                red_v[pl.ds(j * _L, _L)] = acc

            pltpu.sync_copy(red_v, out_hbm.at[task])


_hist_call = functools.partial(
    pl.kernel,
    mesh=plsc.VectorSubcoreMesh(core_axis_name="c", subcore_axis_name="s"),
    out_type=jax.ShapeDtypeStruct((NTASK, NKEY), jnp.float32),
    scratch_types=[
        pltpu.VMEM((2, SC_CHUNK), jnp.int32),
        pltpu.VMEM((HSIZE,), jnp.float32),
        pltpu.VMEM((NKEY,), jnp.float32),
        pltpu.SemaphoreType.DMA,
        pltpu.SemaphoreType.DMA,
    ],
    compiler_params=pltpu.CompilerParams(needs_layout_passes=False),
)(_hist_body)


def _stage3_body(hist_ref, stats_ref, out_ref):
    f32 = jnp.float32
    nr = 4 * NINST

    hist = hist_ref[...]
    hn = hist[:, :K]
    hp = hist[:, K:]
    tri = (lax.broadcasted_iota(jnp.int32, (K, K), 0)
           >= lax.broadcasted_iota(jnp.int32, (K, K), 1)).astype(f32)
    sp = jnp.dot(hp, tri, preferred_element_type=f32)
    sn = jnp.dot(hn, tri, preferred_element_type=f32)

    stats = stats_ref[...]
    cts = stats[:, 0:NINST, 0:1].reshape(nr, 1)
    var_c = stats[:, 0:NINST, 1:2].reshape(nr, 1)
    seed_c = stats[:, 0:NINST, 2:3].reshape(nr, 1)
    bg_col = stats[:, 0:1, 3:4].reshape(4, 1)

    present = cts > 0.0

    def jac(p, n):
        return 1.0 - (cts - p) / jnp.maximum(cts + n, 1e-9)

    d_j = jac(sp, sn) - jac(sp - hp, sn - hn)
    ebar = (lax.broadcasted_iota(jnp.int32, (1, K), 1).astype(f32)
            + 0.5) * (2.0 / K)
    lov = jnp.sum(ebar * d_j, axis=1, keepdims=True)

    z = jnp.where(present, 1.0, 0.0)
    bsel = (lax.broadcasted_iota(jnp.int32, (4, nr), 1) // NINST
            == lax.broadcasted_iota(jnp.int32, (4, nr), 0)).astype(f32)
    inst_l = jnp.dot(bsel, z * lov, preferred_element_type=f32)
    var_l = jnp.dot(bsel, z * var_c, preferred_element_type=f32)
    seed_fg = jnp.dot(bsel, z * seed_c, preferred_element_type=f32)
    obj = jnp.dot(bsel, z, preferred_element_type=f32)

    has = obj > 0.0
    so = jnp.where(has, obj, 1.0)
    inst_l = jnp.where(has, inst_l / so, inst_l)
    var_l = jnp.where(has, var_l / so, var_l)
    seed_total = (bg_col + seed_fg) / (H * W)
    loss_col = inst_l + 10.0 * var_l + seed_total

    out_ref[...] = jnp.sum(loss_col, keepdims=True).reshape(1, 1) * 0.25


def _stage1_call(prediction, instances, labels, off):
    return pl.pallas_call(
        _stage1_body,
        grid=(1,),
        in_specs=[
            pl.BlockSpec((None, 5, H, W), lambda b, o=off: (o, 0, 0, 0)),
            pl.BlockSpec((None, H, W), lambda b, o=off: (o, 0, 0)),
            pl.BlockSpec((None, H, W), lambda b, o=off: (o, 0, 0)),
        ],
        out_specs=[
            pl.BlockSpec((None, NINST, H, W), lambda b: (0, 0, 0, 0)),
            pl.BlockSpec((None, 16, 128), lambda b: (0, 0, 0)),
        ],
        out_shape=[
            jax.ShapeDtypeStruct((1, NINST, H, W), jnp.int32),
            jax.ShapeDtypeStruct((1, 16, 128), jnp.float32),
        ],
    )(prediction, instances, labels)


def kernel(prediction, instances, labels):
    zeros = jnp.zeros((HSIZE,), jnp.float32)
    hists, stats = [], []
    for b in range(4):
        keys_b, stats_b = _stage1_call(prediction, instances, labels, b)
        hists.append(_hist_call(keys_b.reshape(NTASK, NPIX), zeros))
        stats.append(stats_b)

    loss = pl.pallas_call(
        _stage3_body,
        out_shape=jax.ShapeDtypeStruct((1, 1), jnp.float32),
    )(jnp.concatenate(hists, axis=0),
      jnp.concatenate(stats, axis=0))

    return (loss[0, 0], jnp.zeros((), jnp.float32))


# speedup vs baseline: 1.2936x; 1.2936x over previous
import functools

import jax
import jax.numpy as jnp
from jax import lax
from jax.experimental import pallas as pl
from jax.experimental.pallas import tpu as pltpu
from jax.experimental.pallas import tpu_sc as plsc

H = W = 512
K = 2048
NKEY = 2 * K
NPIX = H * W
NINST = 15
NTASK = 2 * NINST

_info = plsc.get_sparse_core_info()
_NC, _NS, _L = _info.num_cores, _info.num_subcores, _info.num_lanes
_NW = _NC * _NS

SC_CHUNK = 16384
SC_NCHUNK = NPIX // SC_CHUNK


def _stage1_body(pred_ref, inst_ref, lab_ref, keys_ref, stats_ref):
    f32 = jnp.float32

    col = lax.broadcasted_iota(jnp.int32, (H, W), 1).astype(f32) * (1.0 / (W - 1))
    row = lax.broadcasted_iota(jnp.int32, (H, W), 0).astype(f32) * (1.0 / (H - 1))
    emb_x = jnp.tanh(pred_ref[0]) + col
    emb_y = jnp.tanh(pred_ref[1]) + row
    sgx = pred_ref[2]
    sgy = pred_ref[3]
    seed = 1.0 / (1.0 + jnp.exp(-pred_ref[4]))
    inst = inst_ref[...]
    lab = lab_ref[...]

    bg_seed = jnp.sum(jnp.where(lab == 0, seed * seed, 0.0))

    si = lax.broadcasted_iota(jnp.int32, (16, 128), 0)
    li = lax.broadcasted_iota(jnp.int32, (16, 128), 1)

    def body(iid, stats):
        mb = inst == iid
        mf = mb.astype(f32)
        cnt = jnp.sum(mf)
        present = cnt > 0.0
        sc = jnp.where(present, cnt, 1.0)
        cx = jnp.sum(mf * emb_x) / sc
        cy = jnp.sum(mf * emb_y) / sc
        sx = jnp.sum(mf * sgx) / sc
        sy = jnp.sum(mf * sgy) / sc
        var_term = (jnp.sum(mf * (sgx - sx) ** 2)
                    + jnp.sum(mf * (sgy - sy) ** 2)) / (2.0 * sc)
        sex = jnp.exp(10.0 * sx)
        sey = jnp.exp(10.0 * sy)
        d = jnp.exp(-((emb_x - cx) ** 2 * sex + (emb_y - cy) ** 2 * sey))
        seed_term = jnp.sum(mf * (seed - d) ** 2)
        e = 1.0 - (d * 2.0 - 1.0) * (mf * 2.0 - 1.0)
        kbin = jnp.clip((e * (K / 2.0)).astype(jnp.int32), 0, K - 1)
        key = kbin + jnp.where(mb, K, 0)
        keys_ref[pl.ds(iid - 1, 1)] = key[None]

        r = iid - 1
        stats = stats + jnp.where((si == r) & (li == 0), cnt, 0.0) \
            + jnp.where((si == r) & (li == 1), var_term, 0.0) \
            + jnp.where((si == r) & (li == 2), seed_term, 0.0)
        return stats

    stats = lax.fori_loop(1, 16, body, jnp.zeros((16, 128), f32))
    stats = stats + jnp.where((si == 0) & (li == 3), bg_seed, 0.0)
    stats_ref[...] = stats


SC_UNROLL = 16
HSTRIDE = NKEY + 1
HSIZE = HSTRIDE * _L


def _hist_body(keys_hbm, zeros_hbm, out_hbm, chunk_v, hist_v, red_v, sem0, sem1):
    wid = lax.axis_index("s") * _NC + lax.axis_index("c")
    lane = lax.iota(jnp.int32, _L)
    loff = lane * HSTRIDE
    ones = jnp.ones((_L,), jnp.float32)
    zeros16 = jnp.zeros((_L,), jnp.float32)
    sems = (sem0, sem1)

    for rnd in range((NTASK + _NW - 1) // _NW):
        task = wid + rnd * _NW

        @pl.when(task < NTASK)
        def _():
            pltpu.sync_copy(zeros_hbm, hist_v)

            cps = {}
            cps[0] = pltpu.async_copy(
                keys_hbm.at[task, pl.ds(0, SC_CHUNK)], chunk_v.at[0], sems[0])
            for ch in range(SC_NCHUNK):
                buf = ch % 2
                if ch + 1 < SC_NCHUNK:
                    nbuf = (ch + 1) % 2
                    cps[ch + 1] = pltpu.async_copy(
                        keys_hbm.at[task, pl.ds((ch + 1) * SC_CHUNK, SC_CHUNK)],
                        chunk_v.at[nbuf], sems[nbuf])
                cps[ch].wait()

                @plsc.parallel_loop(0, SC_CHUNK // _L, 1, unroll=SC_UNROLL)
                def _scatter(j):
                    kk = chunk_v[buf, pl.ds(j * _L, _L)]
                    plsc.addupdate_scatter(hist_v, [kk + loff], ones)

            @plsc.parallel_loop(0, NKEY // _L, 1, unroll=2)
            def _reduce(j):
                acc = zeros16
                for ln in range(_L):
                    iv = lane + (ln * HSTRIDE + j * _L)
                    acc = acc + plsc.load_gather(hist_v, [iv])
                red_v[pl.ds(j * _L, _L)] = acc

            pltpu.sync_copy(red_v, out_hbm.at[task])


_hist_call = functools.partial(
    pl.kernel,
    mesh=plsc.VectorSubcoreMesh(core_axis_name="c", subcore_axis_name="s"),
    out_type=jax.ShapeDtypeStruct((NTASK, NKEY), jnp.float32),
    scratch_types=[
        pltpu.VMEM((2, SC_CHUNK), jnp.int32),
        pltpu.VMEM((HSIZE,), jnp.float32),
        pltpu.VMEM((NKEY,), jnp.float32),
        pltpu.SemaphoreType.DMA,
        pltpu.SemaphoreType.DMA,
    ],
    compiler_params=pltpu.CompilerParams(needs_layout_passes=False),
)(_hist_body)


def _stage3_body(ha_ref, hb_ref, sa_ref, sb_ref, out_ref):
    f32 = jnp.float32
    nr = 4 * NINST

    hist = jnp.concatenate([ha_ref[...], hb_ref[...]], axis=0)
    hn = hist[:, :K]
    hp = hist[:, K:]
    tri = (lax.broadcasted_iota(jnp.int32, (K, K), 0)
           >= lax.broadcasted_iota(jnp.int32, (K, K), 1)).astype(f32)
    sp = jnp.dot(hp, tri, preferred_element_type=f32)
    sn = jnp.dot(hn, tri, preferred_element_type=f32)

    stats = jnp.concatenate([sa_ref[...], sb_ref[...]], axis=0)
    cts = stats[:, 0:NINST, 0:1].reshape(nr, 1)
    var_c = stats[:, 0:NINST, 1:2].reshape(nr, 1)
    seed_c = stats[:, 0:NINST, 2:3].reshape(nr, 1)
    bg_col = stats[:, 0:1, 3:4].reshape(4, 1)

    present = cts > 0.0

    def jac(p, n):
        return 1.0 - (cts - p) / jnp.maximum(cts + n, 1e-9)

    d_j = jac(sp, sn) - jac(sp - hp, sn - hn)
    ebar = (lax.broadcasted_iota(jnp.int32, (1, K), 1).astype(f32)
            + 0.5) * (2.0 / K)
    lov = jnp.sum(ebar * d_j, axis=1, keepdims=True)

    z = jnp.where(present, 1.0, 0.0)
    bsel = (lax.broadcasted_iota(jnp.int32, (4, nr), 1) // NINST
            == lax.broadcasted_iota(jnp.int32, (4, nr), 0)).astype(f32)
    inst_l = jnp.dot(bsel, z * lov, preferred_element_type=f32)
    var_l = jnp.dot(bsel, z * var_c, preferred_element_type=f32)
    seed_fg = jnp.dot(bsel, z * seed_c, preferred_element_type=f32)
    obj = jnp.dot(bsel, z, preferred_element_type=f32)

    has = obj > 0.0
    so = jnp.where(has, obj, 1.0)
    inst_l = jnp.where(has, inst_l / so, inst_l)
    var_l = jnp.where(has, var_l / so, var_l)
    seed_total = (bg_col + seed_fg) / (H * W)
    loss_col = inst_l + 10.0 * var_l + seed_total

    out_ref[...] = jnp.sum(loss_col, keepdims=True).reshape(1, 1) * 0.25


def _stage1_call(prediction, instances, labels, off):
    return pl.pallas_call(
        _stage1_body,
        grid=(2,),
        in_specs=[
            pl.BlockSpec((None, 5, H, W), lambda b, o=off: (b + o, 0, 0, 0)),
            pl.BlockSpec((None, H, W), lambda b, o=off: (b + o, 0, 0)),
            pl.BlockSpec((None, H, W), lambda b, o=off: (b + o, 0, 0)),
        ],
        out_specs=[
            pl.BlockSpec((None, NINST, H, W), lambda b: (b, 0, 0, 0)),
            pl.BlockSpec((None, 16, 128), lambda b: (b, 0, 0)),
        ],
        out_shape=[
            jax.ShapeDtypeStruct((2, NINST, H, W), jnp.int32),
            jax.ShapeDtypeStruct((2, 16, 128), jnp.float32),
        ],
    )(prediction, instances, labels)


def kernel(prediction, instances, labels):
    keys_a, stats_a = _stage1_call(prediction, instances, labels, 0)
    keys_b, stats_b = _stage1_call(prediction, instances, labels, 2)

    zeros = jnp.zeros((HSIZE,), jnp.float32)
    hist_a = _hist_call(keys_a.reshape(NTASK, NPIX), zeros)
    hist_b = _hist_call(keys_b.reshape(NTASK, NPIX), zeros)

    loss = pl.pallas_call(
        _stage3_body,
        out_shape=jax.ShapeDtypeStruct((1, 1), jnp.float32),
    )(hist_a, hist_b, stats_a, stats_b)

    return (loss[0, 0], jnp.zeros((), jnp.float32))
